# baseline (speedup 1.0000x reference)
import functools

import jax
import jax.numpy as jnp
from jax import lax
from jax.experimental import pallas as pl
from jax.experimental.pallas import tpu as pltpu
from jax.experimental.pallas import tpu_sc as plsc

B = 16384
V = 1000000
D = 64
D2 = 128

_CBLK = 16384
_QBLK = _CBLK // 4
_CLOG = _CBLK.bit_length() - 1
_QLOG = _QBLK.bit_length() - 1
_NSTEP = (V + _CBLK - 1) // _CBLK
_VROWS = _NSTEP * _QBLK

_NC = 2
_NS = 16
_NW = _NC * _NS
_BPW = B // _NW
_CHUNK = 256
_NCHUNK = _BPW // _CHUNK
_NBUF = 3
_NSTAGE = 4 * _NCHUNK


def _pack_one(x_ref, eye_ref):
    t = lax.dot_general(x_ref[...].astype(jnp.bfloat16), eye_ref[...],
                        (((0,), (0,)), ((), ())),
                        preferred_element_type=jnp.float32)
    u = lax.bitcast_convert_type(t, jnp.int32)
    q = [u[i * _QBLK:(i + 1) * _QBLK] for i in range(4)]
    mhi = jnp.int32(-65536)
    w01 = lax.shift_right_logical(q[0], 16) | (q[1] & mhi)
    w23 = lax.shift_right_logical(q[2], 16) | (q[3] & mhi)
    return jnp.concatenate([w01, w23], axis=1)


def _tc_pack_body(ig, um, im, eye, o_ig, o_um, o_im):
    o_ig[...] = _pack_one(ig, eye)
    o_um[...] = _pack_one(um, eye)
    o_im[...] = _pack_one(im, eye)


def _tc_pack(t_ig, t_um, t_im):
    eye = jnp.eye(D, dtype=jnp.bfloat16)
    in_spec = pl.BlockSpec((D, _CBLK), lambda j: (0, j))
    out_spec = pl.BlockSpec((_QBLK, D2), lambda j: (j, 0))
    return pl.pallas_call(
        _tc_pack_body,
        grid=(_NSTEP,),
        in_specs=[in_spec, in_spec, in_spec,
                  pl.BlockSpec((D, D), lambda j: (0, 0))],
        out_specs=[out_spec] * 3,
        out_shape=[jax.ShapeDtypeStruct((_VROWS, D2), jnp.int32)] * 3,
        compiler_params=pltpu.CompilerParams(
            fuse_transposed_lhs_in_matmul=True,
            vmem_limit_bytes=100 * 1024 * 1024,
        ),
    )(t_ig, t_um, t_im, eye)


def _sc_gather_body(idxg_hbm, idxu_hbm, idxi_hbm, ug_hbm, ig_hbm, um_hbm,
                    im_hbm, o_ug, o_ig, o_um, o_im,
                    idx_g, idx_u, idx_i, bufs, gsems, wsems):
    wid = lax.axis_index("s") * _NC + lax.axis_index("c")
    base = wid * _BPW
    pltpu.sync_copy(idxg_hbm.at[pl.ds(base, _BPW)], idx_g)
    pltpu.sync_copy(idxu_hbm.at[pl.ds(base, _BPW)], idx_u)
    pltpu.sync_copy(idxi_hbm.at[pl.ds(base, _BPW)], idx_i)

    srcs = ((ug_hbm, idx_g), (ig_hbm, idx_i), (um_hbm, idx_u), (im_hbm, idx_i))
    outs = (o_ug, o_ig, o_um, o_im)

    def gather(s):
        tbl, idx = srcs[s // _NCHUNK]
        c = s % _NCHUNK
        k = s % _NBUF
        return pltpu.async_copy(tbl.at[idx.at[pl.ds(c * _CHUNK, _CHUNK)]],
                                bufs.at[k], gsems.at[k])

    def writeback(s):
        out = outs[s // _NCHUNK]
        c = s % _NCHUNK
        k = s % _NBUF
        row0 = base + c * _CHUNK
        return pltpu.async_copy(bufs.at[k], out.at[pl.ds(row0, _CHUNK)],
                                wsems.at[k])

    caps_g = {}
    caps_w = {}
    for s in range(min(_NBUF, _NSTAGE)):
        caps_g[s] = gather(s)
    for s in range(_NSTAGE):
        caps_g[s].wait()
        caps_w[s] = writeback(s)
        nxt = s + _NBUF
        if nxt < _NSTAGE:
            caps_w[s].wait()
            caps_g[nxt] = gather(nxt)
    for s in range(max(0, _NSTAGE - _NBUF), _NSTAGE):
        if s in caps_w:
            caps_w[s].wait()


_sc_gather = functools.partial(
    pl.kernel,
    out_type=[jax.ShapeDtypeStruct((B, D2), jnp.int32)] * 4,
    mesh=plsc.VectorSubcoreMesh(core_axis_name="c", subcore_axis_name="s"),
    compiler_params=pltpu.CompilerParams(use_tc_tiling_on_sc=True),
    scratch_types=[
        pltpu.VMEM((_BPW,), jnp.int32),
        pltpu.VMEM((_BPW,), jnp.int32),
        pltpu.VMEM((_BPW,), jnp.int32),
        pltpu.VMEM((_NBUF, _CHUNK, D2), jnp.int32),
        pltpu.SemaphoreType.DMA((_NBUF,)),
        pltpu.SemaphoreType.DMA((_NBUF,)),
    ],
)(_sc_gather_body)


_BLK = 2048


def _unpack(x, vid):
    colhi = (vid & (2 * _QBLK)) == (2 * _QBLK)
    xx = jnp.where(colhi, x[:, D:], x[:, :D])
    hi16 = (vid & _QBLK) == _QBLK
    half = jnp.where(hi16, lax.shift_right_logical(xx, 16), xx)
    return lax.bitcast_convert_type((half & 0xFFFF) << 16, jnp.float32)


def _tc_dense_body(uid, iid, ug2, ig2, um2, im2,
                   w1a, w1b, b1, w2, b2, w3, b3, wog, woh, bo, out):
    f32 = jnp.float32
    uv = uid[...]
    iv = iid[...]
    ugf = lax.bitcast_convert_type(ug2[...], jnp.float32)
    ug = jnp.where((uv & 1) == 1, ugf[:, D:], ugf[:, :D])
    ig = _unpack(ig2[...], iv)
    um = _unpack(um2[...], uv)
    im = _unpack(im2[...], iv)
    gmf = ug * ig
    h = jnp.dot(um, w1a[...], preferred_element_type=f32)
    h += jnp.dot(im, w1b[...], preferred_element_type=f32)
    h = jnp.maximum(h + b1[...], 0.0)
    h = jnp.maximum(jnp.dot(h, w2[...], preferred_element_type=f32) + b2[...], 0.0)
    h = jnp.maximum(jnp.dot(h, w3[...], preferred_element_type=f32) + b3[...], 0.0)
    logit = jnp.sum(gmf * wog[...], axis=1) + jnp.sum(h * woh[...], axis=1)
    out[...] = jax.nn.sigmoid(logit + bo[0, 0])


def _tc_dense(uid, iid, ug2, ig2, um2, im2,
              w1a, w1b, b1, w2, b2, w3, b3, wog, woh, bo):
    n_blk = B // _BLK
    id_spec = pl.BlockSpec((_BLK, 1), lambda i: (i, 0))
    row_spec = pl.BlockSpec((_BLK, D2), lambda i: (i, 0))
    full = lambda a: pl.BlockSpec(a.shape, lambda i: (0,) * a.ndim)
    return pl.pallas_call(
        _tc_dense_body,
        grid=(n_blk,),
        in_specs=[id_spec, id_spec, row_spec, row_spec, row_spec, row_spec,
                  full(w1a), full(w1b), full(b1), full(w2), full(b2),
                  full(w3), full(b3), full(wog), full(woh), full(bo)],
        out_specs=pl.BlockSpec((_BLK,), lambda i: (i,)),
        out_shape=jax.ShapeDtypeStruct((B,), jnp.float32),
    )(uid, iid, ug2, ig2, um2, im2,
      w1a, w1b, b1, w2, b2, w3, b3, wog, woh, bo)


def kernel(user_ids, item_ids, ue_gmf, ie_gmf, ue_mlp, ie_mlp,
           W1, b1, W2, b2, W3, b3, Wo, bo):
    p_ig, p_um, p_im = _tc_pack(ie_gmf.T, ue_mlp.T, ie_mlp.T)
    p_ug = lax.bitcast_convert_type(ue_gmf.reshape(V // 2, D2), jnp.int32)
    ig_row = lax.shift_right_logical(user_ids, 1)
    iu_row = ((lax.shift_right_logical(user_ids, _CLOG) << _QLOG)
              | (user_ids & (_QBLK - 1)))
    ii_row = ((lax.shift_right_logical(item_ids, _CLOG) << _QLOG)
              | (item_ids & (_QBLK - 1)))
    ug2, ig2, um2, im2 = _sc_gather(ig_row, iu_row, ii_row,
                                    p_ug, p_ig, p_um, p_im)
    w1a, w1b = W1[:D], W1[D:]
    wog = Wo[:D, 0].reshape(1, D)
    woh = Wo[D:, 0].reshape(1, Wo.shape[0] - D)
    return _tc_dense(user_ids.reshape(B, 1), item_ids.reshape(B, 1),
                     ug2, ig2, um2, im2,
                     w1a, w1b, b1.reshape(1, -1), W2, b2.reshape(1, -1),
                     W3, b3.reshape(1, -1), wog, woh, bo.reshape(1, 1))


# speedup vs baseline: 1.9685x; 1.9685x over previous
import functools

import jax
import jax.numpy as jnp
from jax import lax
from jax.experimental import pallas as pl
from jax.experimental.pallas import tpu as pltpu
from jax.experimental.pallas import tpu_sc as plsc

B = 16384
V = 1000000
D = 64
D2 = 128

_CBLK = 16384
_QBLK = _CBLK // 4
_CLOG = _CBLK.bit_length() - 1
_QLOG = _QBLK.bit_length() - 1
_NSTEP = (V + _CBLK - 1) // _CBLK
_VROWS = _NSTEP * _QBLK

_NC = 2
_NS = 16
_NW = _NC * _NS
_BPW = B // _NW
_CHUNK = 256
_NCHUNK = _BPW // _CHUNK
_NBUF = 3
_NSTAGE = 4 * _NCHUNK


def _pack_one(x_ref, eye_ref):
    t = lax.dot_general(x_ref[...].astype(jnp.bfloat16), eye_ref[...],
                        (((0,), (0,)), ((), ())),
                        preferred_element_type=jnp.float32)
    u = lax.bitcast_convert_type(t, jnp.int32)
    q = [u[i * _QBLK:(i + 1) * _QBLK] for i in range(4)]
    mhi = jnp.int32(-65536)
    w01 = lax.shift_right_logical(q[0], 16) | (q[1] & mhi)
    w23 = lax.shift_right_logical(q[2], 16) | (q[3] & mhi)
    return jnp.concatenate([w01, w23], axis=1)


def _tc_pack_body(ug, ig, um, im, eye, o_ug, o_ig, o_um, o_im):
    o_ug[...] = _pack_one(ug, eye)
    o_ig[...] = _pack_one(ig, eye)
    o_um[...] = _pack_one(um, eye)
    o_im[...] = _pack_one(im, eye)


def _tc_pack(t_ug, t_ig, t_um, t_im):
    eye = jnp.eye(D, dtype=jnp.bfloat16)
    in_spec = pl.BlockSpec((D, _CBLK), lambda j: (0, j))
    out_spec = pl.BlockSpec((_QBLK, D2), lambda j: (j, 0))
    return pl.pallas_call(
        _tc_pack_body,
        grid=(_NSTEP,),
        in_specs=[in_spec, in_spec, in_spec, in_spec,
                  pl.BlockSpec((D, D), lambda j: (0, 0))],
        out_specs=[out_spec] * 4,
        out_shape=[jax.ShapeDtypeStruct((_VROWS, D2), jnp.int32)] * 4,
        compiler_params=pltpu.CompilerParams(
            fuse_transposed_lhs_in_matmul=True,
            vmem_limit_bytes=100 * 1024 * 1024,
        ),
    )(t_ug, t_ig, t_um, t_im, eye)


def _sc_gather_body(idxu_hbm, idxi_hbm, ug_hbm, ig_hbm, um_hbm, im_hbm,
                    o_ug, o_ig, o_um, o_im,
                    idx_u, idx_i, bufs, gsems, wsems):
    wid = lax.axis_index("s") * _NC + lax.axis_index("c")
    base = wid * _BPW
    pltpu.sync_copy(idxu_hbm.at[pl.ds(base, _BPW)], idx_u)
    pltpu.sync_copy(idxi_hbm.at[pl.ds(base, _BPW)], idx_i)

    srcs = ((ug_hbm, idx_u), (ig_hbm, idx_i), (um_hbm, idx_u), (im_hbm, idx_i))
    outs = (o_ug, o_ig, o_um, o_im)

    def gather(s):
        tbl, idx = srcs[s // _NCHUNK]
        c = s % _NCHUNK
        k = s % _NBUF
        return pltpu.async_copy(tbl.at[idx.at[pl.ds(c * _CHUNK, _CHUNK)]],
                                bufs.at[k], gsems.at[k])

    def writeback(s):
        out = outs[s // _NCHUNK]
        c = s % _NCHUNK
        k = s % _NBUF
        row0 = base + c * _CHUNK
        return pltpu.async_copy(bufs.at[k], out.at[pl.ds(row0, _CHUNK)],
                                wsems.at[k])

    caps_g = {}
    caps_w = {}
    for s in range(min(_NBUF, _NSTAGE)):
        caps_g[s] = gather(s)
    for s in range(_NSTAGE):
        caps_g[s].wait()
        caps_w[s] = writeback(s)
        nxt = s + _NBUF
        if nxt < _NSTAGE:
            caps_w[s].wait()
            caps_g[nxt] = gather(nxt)
    for s in range(max(0, _NSTAGE - _NBUF), _NSTAGE):
        if s in caps_w:
            caps_w[s].wait()


_sc_gather = functools.partial(
    pl.kernel,
    out_type=[jax.ShapeDtypeStruct((B, D2), jnp.int32)] * 4,
    mesh=plsc.VectorSubcoreMesh(core_axis_name="c", subcore_axis_name="s"),
    compiler_params=pltpu.CompilerParams(use_tc_tiling_on_sc=True),
    scratch_types=[
        pltpu.VMEM((_BPW,), jnp.int32),
        pltpu.VMEM((_BPW,), jnp.int32),
        pltpu.VMEM((_NBUF, _CHUNK, D2), jnp.int32),
        pltpu.SemaphoreType.DMA((_NBUF,)),
        pltpu.SemaphoreType.DMA((_NBUF,)),
    ],
)(_sc_gather_body)


_BLK = 2048


def _unpack(x, vid):
    colhi = (vid & (2 * _QBLK)) == (2 * _QBLK)
    xx = jnp.where(colhi, x[:, D:], x[:, :D])
    hi16 = (vid & _QBLK) == _QBLK
    half = jnp.where(hi16, lax.shift_right_logical(xx, 16), xx)
    return lax.bitcast_convert_type((half & 0xFFFF) << 16, jnp.float32)


def _tc_dense_body(uid, iid, ug2, ig2, um2, im2,
                   w1a, w1b, b1, w2, b2, w3, b3, wog, woh, bo, out):
    f32 = jnp.float32
    uv = uid[...]
    iv = iid[...]
    ug = _unpack(ug2[...], uv)
    ig = _unpack(ig2[...], iv)
    um = _unpack(um2[...], uv)
    im = _unpack(im2[...], iv)
    gmf = ug * ig
    h = jnp.dot(um, w1a[...], preferred_element_type=f32)
    h += jnp.dot(im, w1b[...], preferred_element_type=f32)
    h = jnp.maximum(h + b1[...], 0.0)
    h = jnp.maximum(jnp.dot(h, w2[...], preferred_element_type=f32) + b2[...], 0.0)
    h = jnp.maximum(jnp.dot(h, w3[...], preferred_element_type=f32) + b3[...], 0.0)
    logit = jnp.sum(gmf * wog[...], axis=1) + jnp.sum(h * woh[...], axis=1)
    out[...] = jax.nn.sigmoid(logit + bo[0, 0])


def _tc_dense(uid, iid, ug2, ig2, um2, im2,
              w1a, w1b, b1, w2, b2, w3, b3, wog, woh, bo):
    n_blk = B // _BLK
    id_spec = pl.BlockSpec((_BLK, 1), lambda i: (i, 0))
    row_spec = pl.BlockSpec((_BLK, D2), lambda i: (i, 0))
    full = lambda a: pl.BlockSpec(a.shape, lambda i: (0,) * a.ndim)
    return pl.pallas_call(
        _tc_dense_body,
        grid=(n_blk,),
        in_specs=[id_spec, id_spec, row_spec, row_spec, row_spec, row_spec,
                  full(w1a), full(w1b), full(b1), full(w2), full(b2),
                  full(w3), full(b3), full(wog), full(woh), full(bo)],
        out_specs=pl.BlockSpec((_BLK,), lambda i: (i,)),
        out_shape=jax.ShapeDtypeStruct((B,), jnp.float32),
    )(uid, iid, ug2, ig2, um2, im2,
      w1a, w1b, b1, w2, b2, w3, b3, wog, woh, bo)


def kernel(user_ids, item_ids, ue_gmf, ie_gmf, ue_mlp, ie_mlp,
           W1, b1, W2, b2, W3, b3, Wo, bo):
    p_ug, p_ig, p_um, p_im = _tc_pack(ue_gmf.T, ie_gmf.T, ue_mlp.T, ie_mlp.T)
    iu_row = ((lax.shift_right_logical(user_ids, _CLOG) << _QLOG)
              | (user_ids & (_QBLK - 1)))
    ii_row = ((lax.shift_right_logical(item_ids, _CLOG) << _QLOG)
              | (item_ids & (_QBLK - 1)))
    ug2, ig2, um2, im2 = _sc_gather(iu_row, ii_row, p_ug, p_ig, p_um, p_im)
    w1a, w1b = W1[:D], W1[D:]
    wog = Wo[:D, 0].reshape(1, D)
    woh = Wo[D:, 0].reshape(1, Wo.shape[0] - D)
    return _tc_dense(user_ids.reshape(B, 1), item_ids.reshape(B, 1),
                     ug2, ig2, um2, im2,
                     w1a, w1b, b1.reshape(1, -1), W2, b2.reshape(1, -1),
                     W3, b3.reshape(1, -1), wog, woh, bo.reshape(1, 1))
